# Initial kernel scaffold; baseline (speedup 1.0000x reference)
import jax
import jax.numpy as jnp
from jax.experimental import pallas as pl


def kernel(inp, matrix):
    raise NotImplementedError("write your pallas kernel here")


# speedup vs baseline: 1.0339x; 1.0339x over previous
import jax
import jax.numpy as jnp
from jax.experimental import pallas as pl
from jax.experimental.pallas import tpu as pltpu

_BM = 512


def _matmul_block(mat_ref, inp_ref, out_ref):
    out_ref[...] = jax.lax.dot_general(
        mat_ref[...],
        inp_ref[...],
        dimension_numbers=(((1,), (0,)), ((), ())),
        preferred_element_type=jnp.float32,
    )


def kernel(inp, matrix):
    B, C, S = inp.shape
    M = matrix.shape[0]
    inp_flat = inp.reshape(B * C, S)

    out_flat = pl.pallas_call(
        _matmul_block,
        grid=(M // _BM,),
        in_specs=[
            pl.BlockSpec((_BM, matrix.shape[1]), lambda i: (i, 0)),
            pl.BlockSpec((B * C, S), lambda i: (0, 0)),
        ],
        out_specs=pl.BlockSpec((_BM, S), lambda i: (i, 0)),
        out_shape=jax.ShapeDtypeStruct((M, S), jnp.float32),
        compiler_params=pltpu.CompilerParams(
            dimension_semantics=("arbitrary",),
        ),
    )(matrix, inp_flat)

    return out_flat.reshape(B, C, S)
